# baseline (speedup 1.0000x reference)
import jax
import jax.numpy as jnp
from jax.experimental import pallas as pl

_TEMPERATURE = 64.0
_Q_BLOCK = 120


def _protonet_body(s_ref, q_ref, o_ref):
    proto = jnp.sum(s_ref[...], axis=0) * (1.0 / s_ref.shape[0])
    q = q_ref[...]
    qn = jnp.sum(q * q, axis=1, keepdims=True)
    pn = jnp.sum(proto * proto, axis=1)[None, :]
    cross = jax.lax.dot_general(
        q, proto, (((1,), (1,)), ((), ())),
        preferred_element_type=jnp.float32,
    )
    o_ref[...] = (2.0 * cross - qn - pn) * (1.0 / _TEMPERATURE)


def kernel(support, query):
    n_batch, n_shot, n_way, emb_dim = support.shape
    n_query = n_batch * query.shape[1] * n_way
    s = support.reshape(n_shot, n_way, emb_dim)
    q = query.reshape(n_query, emb_dim)
    grid = n_query // _Q_BLOCK
    return pl.pallas_call(
        _protonet_body,
        grid=(grid,),
        in_specs=[
            pl.BlockSpec((n_shot, n_way, emb_dim), lambda i: (0, 0, 0)),
            pl.BlockSpec((_Q_BLOCK, emb_dim), lambda i: (i, 0)),
        ],
        out_specs=pl.BlockSpec((_Q_BLOCK, n_way), lambda i: (i, 0)),
        out_shape=jax.ShapeDtypeStruct((n_query, n_way), jnp.float32),
    )(s, q)


# speedup vs baseline: 1.6409x; 1.6409x over previous
import jax
import jax.numpy as jnp
from jax.experimental import pallas as pl

_TEMPERATURE = 64.0
_Q_BLOCK = 120


def _protonet_body(s_ref, q_ref, o_ref):
    proto = jnp.sum(s_ref[...], axis=0) * (1.0 / s_ref.shape[0])
    q = q_ref[...]
    qn = jnp.sum(q * q, axis=1, keepdims=True)
    pn = jnp.sum(proto * proto, axis=1)[None, :]
    cross = jax.lax.dot_general(
        q, proto, (((1,), (1,)), ((), ())),
        preferred_element_type=jnp.float32,
    )
    o_ref[...] = (2.0 * cross - qn - pn) * (1.0 / _TEMPERATURE)


def kernel(support, query):
    n_batch, n_shot, n_way, emb_dim = support.shape
    n_query = n_batch * query.shape[1] * n_way
    s = support.reshape(n_shot, n_way, emb_dim)
    q = query.reshape(n_query, emb_dim)
    return pl.pallas_call(
        _protonet_body,
        out_shape=jax.ShapeDtypeStruct((n_query, n_way), jnp.float32),
    )(s, q)
